# baseline (speedup 1.0000x reference)
import functools

import jax
import jax.numpy as jnp
from jax import lax
from jax.experimental import pallas as pl
from jax.experimental.pallas import tpu as pltpu
from jax.experimental.pallas import tpu_sc as plsc

_NBINS = (128, 64, 32)
_NH_TOT = 224
_NW_TOT = 221
_N_CUTS = 131072
_N_GENES = 5000
_NG_PAD = 5120
_TBL_COLS = 464
_D_COLS = 232
_UH_OFF = (0, 128, 192)
_W_OFF = (224, 351, 414)
_NWORKERS = 32
_B = 64


def _table_kernel(uh_ref, uw_ref, out_ref):
    uh = uh_ref[:]
    uw = uw_ref[:]
    parts = [uh]
    ow = 0
    for n in _NBINS:
        nw = n - 1
        u = uw[:, ow:ow + nw]
        m = jnp.max(u, axis=-1, keepdims=True)
        e = jnp.exp(u - m)
        s = jnp.sum(e, axis=-1, keepdims=True)
        parts.append(e / s)
        ow += nw
    parts.append(jnp.zeros((uh.shape[0], _TBL_COLS - _NH_TOT - _NW_TOT), jnp.float32))
    out_ref[:] = jnp.concatenate(parts, axis=-1)


_FULL_COMPUTE = True

_LN2 = 0.6931471805599453
_SQRT2 = 1.4142135623730951


def _log_f32(y):
    bits = lax.bitcast_convert_type(y, jnp.int32)
    m = lax.bitcast_convert_type(
        jnp.bitwise_or(jnp.bitwise_and(bits, 0x007FFFFF), 0x3F800000), jnp.float32)
    e = jnp.right_shift(bits, 23) - 127
    big = m >= _SQRT2
    m = jnp.where(big, m * 0.5, m)
    ef = (e + jnp.where(big, 1, 0)).astype(jnp.float32)
    s = (m - 1.0) / (m + 1.0)
    s2 = s * s
    p = (1.0 / 9.0)
    p = p * s2 + (1.0 / 7.0)
    p = p * s2 + 0.2
    p = p * s2 + (1.0 / 3.0)
    p = p * s2 + 1.0
    return ef * _LN2 + 2.0 * s * p


def _splat_i32(v):
    return jnp.zeros((16,), jnp.int32) + v


def _level_multi(xs, cvecs, rows_v, delta_v, lvl):
    n = len(xs)
    nh = _NBINS[lvl]
    nb = nh - 1
    uh0 = _UH_OFF[lvl]
    w0 = _W_OFF[lvl]
    d0 = _UH_OFF[lvl]

    def e_at(cvec, k):
        cu = plsc.load_gather(rows_v, [cvec, _splat_i32(uh0 + k)])
        cd = plsc.load_gather(delta_v, [cvec, _splat_i32(d0 + k)])
        return jnp.exp(cu + cd)

    def w_at(cvec, j):
        return plsc.load_gather(rows_v, [cvec, _splat_i32(w0 + j)])

    zero = jnp.zeros((16,), jnp.float32)
    one = jnp.ones((16,), jnp.float32)

    def step(k, st, i, last):
        loc, area, cw, cloc, cle, cre, ccdf, ep = st
        e = e_at(cvecs[i], k)
        wj = w_at(cvecs[i], k - 1)
        t = (ep + e) * 0.5 * wj
        locn = loc + wj
        if last:
            sel = xs[i] >= loc
        else:
            sel = (xs[i] >= loc) & (xs[i] < locn)
        cw = jnp.where(sel, wj, cw)
        cloc = jnp.where(sel, loc, cloc)
        cle = jnp.where(sel, ep, cle)
        cre = jnp.where(sel, e, cre)
        ccdf = jnp.where(sel, area, ccdf)
        return (locn, area + t, cw, cloc, cle, cre, ccdf, e)

    def body(k, carry):
        out = []
        for i in range(n):
            out.extend(step(k, carry[8 * i:8 * i + 8], i, False))
        return tuple(out)

    carry = []
    for i in range(n):
        carry.extend((zero, zero, one, zero, one, one, zero, e_at(cvecs[i], 0)))
    carry = lax.fori_loop(1, nh - 1, body, tuple(carry), unroll=16)

    outs, lads = [], []
    for i in range(n):
        st = step(nh - 1, carry[8 * i:8 * i + 8], i, True)
        _, area, cw, cloc, cle, cre, ccdf, _ = st
        inv = 1.0 / area
        lh = cle * inv
        rh = cre * inv
        lcdf = ccdf * inv
        alpha = (xs[i] - cloc) / cw
        dh = rh - lh
        out = (0.5 * dh * cw) * alpha * alpha + (lh * cw) * alpha + lcdf
        outs.append(jnp.clip(out, 0.0, 1.0))
        lads.append(_log_f32(alpha * dh + lh))
    return outs, lads


def _sc_body(x_hbm, lgi_hbm, delta_hbm, table_hbm, genes_hbm,
             outx_hbm, outlad_hbm,
             genes_v, lgi_all, x_all, gidx_a, gidx_b,
             delta_a, delta_b, rows_a, rows_b, ox_all, ol_all, sem_a, sem_b):
    wid = lax.axis_index("s") * 2 + lax.axis_index("c")
    cuts_per = _N_CUTS // _NWORKERS
    ngroups = cuts_per // _B
    gidx_v = (gidx_a, gidx_b)
    delta_v = (delta_a, delta_b)
    rows_v = (rows_a, rows_b)
    sems = (sem_a, sem_b)
    tec_base = wid * cuts_per
    pltpu.sync_copy(genes_hbm, genes_v)
    pltpu.sync_copy(lgi_hbm.at[pl.ds(tec_base, cuts_per)], lgi_all)
    pltpu.sync_copy(x_hbm.at[pl.ds(tec_base, cuts_per)], x_all)
    lane = lax.iota(jnp.int32, 16)

    def prefetch(gi, b):
        base = tec_base + gi * _B
        for sub in range(_B // 16):
            g16 = lgi_all[pl.ds(gi * _B + sub * 16, 16)]
            gg = plsc.load_gather(genes_v, [jnp.right_shift(g16, 4),
                                            jnp.bitwise_and(g16, 15)])
            gidx_v[b][pl.ds(sub * 16, 16)] = plsc.bitcast(gg, jnp.int32)
        pltpu.async_copy(delta_hbm.at[pl.ds(base, _B)], delta_v[b], sems[b])
        pltpu.async_copy(table_hbm.at[gidx_v[b]], rows_v[b], sems[b])

    def wait_bufs(gi, b):
        base = tec_base + gi * _B
        pltpu.make_async_copy(delta_hbm.at[pl.ds(base, _B)], delta_v[b],
                              sems[b]).wait()
        pltpu.make_async_copy(table_hbm.at[gidx_v[b]], rows_v[b],
                              sems[b]).wait()

    prefetch(0, 0)

    def outer(gg, dummy):
        for b in range(2):
            gi = 2 * gg + b

            @pl.when(gi + 1 < ngroups)
            def _():
                prefetch(gi + 1, 1 - b)

            wait_bufs(gi, b)
            for pair in range(_B // 16):
                subs = (pair,)
                offs = [gi * _B + s * 16 for s in subs]
                cvecs = [lane + s * 16 for s in subs]
                xs = [x_all[pl.ds(off, 16)] for off in offs]
                lads = [jnp.zeros((16,), jnp.float32) for _ in subs]
                if _FULL_COMPUTE:
                    for lvl in range(3):
                        xs, ls = _level_multi(xs, cvecs, rows_v[b],
                                              delta_v[b], lvl)
                        lads = [a + c for a, c in zip(lads, ls)]
                else:
                    xs = [x + plsc.load_gather(rows_v[b], [cv, _splat_i32(0)])
                          for x, cv in zip(xs, cvecs)]
                    lads = [a + plsc.load_gather(delta_v[b], [cv, _splat_i32(0)])
                            for a, cv in zip(lads, cvecs)]
                for i, off in enumerate(offs):
                    ox_all[pl.ds(off, 16)] = xs[i]
                    ol_all[pl.ds(off, 16)] = lads[i]
        return dummy

    lax.fori_loop(0, ngroups // 2, outer, jnp.int32(0))
    pltpu.sync_copy(ox_all, outx_hbm.at[pl.ds(tec_base, cuts_per)])
    pltpu.sync_copy(ol_all, outlad_hbm.at[pl.ds(tec_base, cuts_per)])


@functools.lru_cache(maxsize=1)
def _get_sc_kernel():
    return functools.partial(
        pl.kernel,
        mesh=plsc.VectorSubcoreMesh(core_axis_name="c", subcore_axis_name="s"),
        compiler_params=pltpu.CompilerParams(use_tc_tiling_on_sc=False,
                                             needs_layout_passes=False),
        out_type=[jax.ShapeDtypeStruct((_N_CUTS,), jnp.float32),
                  jax.ShapeDtypeStruct((_N_CUTS,), jnp.float32)],
        scratch_types=[
            pltpu.VMEM((32, 16), jnp.float32),
            pltpu.VMEM((_N_CUTS // _NWORKERS,), jnp.int32),
            pltpu.VMEM((_N_CUTS // _NWORKERS,), jnp.float32),
            pltpu.VMEM((_B,), jnp.int32),
            pltpu.VMEM((_B,), jnp.int32),
            pltpu.VMEM((_B, _D_COLS), jnp.float32),
            pltpu.VMEM((_B, _D_COLS), jnp.float32),
            pltpu.VMEM((_B, _TBL_COLS), jnp.float32),
            pltpu.VMEM((_B, _TBL_COLS), jnp.float32),
            pltpu.VMEM((_N_CUTS // _NWORKERS,), jnp.float32),
            pltpu.VMEM((_N_CUTS // _NWORKERS,), jnp.float32),
            pltpu.SemaphoreType.DMA,
            pltpu.SemaphoreType.DMA,
        ],
    )(_sc_body)


def kernel(x, genes_oi, local_gene_ix, delta, unnormalized_heights, unnormalized_widths):
    uh = jnp.pad(unnormalized_heights, ((0, _NG_PAD - _N_GENES), (0, 0)))
    uw = jnp.pad(unnormalized_widths, ((0, _NG_PAD - _N_GENES), (0, _NH_TOT - _NW_TOT)))
    table = pl.pallas_call(
        _table_kernel,
        grid=(_NG_PAD // 512,),
        in_specs=[pl.BlockSpec((512, _NH_TOT), lambda i: (i, 0)),
                  pl.BlockSpec((512, _NH_TOT), lambda i: (i, 0))],
        out_specs=pl.BlockSpec((512, _TBL_COLS), lambda i: (i, 0)),
        out_shape=jax.ShapeDtypeStruct((_NG_PAD, _TBL_COLS), jnp.float32),
    )(uh, uw)
    genes_pad = lax.bitcast_convert_type(
        jnp.pad(genes_oi, (0, 512 - 500)).reshape(32, 16), jnp.float32)
    delta_p = jnp.pad(delta, ((0, 0), (0, _D_COLS - _NH_TOT)))
    outx, outlad = _get_sc_kernel()(x, local_gene_ix, delta_p, table, genes_pad)
    return outx, outlad


# speedup vs baseline: 1.3178x; 1.3178x over previous
import functools

import jax
import jax.numpy as jnp
from jax import lax
from jax.experimental import pallas as pl
from jax.experimental.pallas import tpu as pltpu
from jax.experimental.pallas import tpu_sc as plsc

_NBINS = (128, 64, 32)
_NH_TOT = 224
_NW_TOT = 221
_N_CUTS = 131072
_N_GENES = 5000
_NG_PAD = 5120
_TBL_COLS = 464
_D_COLS = 232
_UH_OFF = (0, 128, 192)
_W_OFF = (224, 351, 414)
_NWORKERS = 32
_B = 64


def _table_kernel(uh_ref, uw_ref, out_ref):
    uh = uh_ref[:]
    uw = uw_ref[:]
    parts = [uh]
    ow = 0
    for n in _NBINS:
        nw = n - 1
        u = uw[:, ow:ow + nw]
        m = jnp.max(u, axis=-1, keepdims=True)
        e = jnp.exp(u - m)
        s = jnp.sum(e, axis=-1, keepdims=True)
        parts.append(e / s)
        ow += nw
    parts.append(jnp.zeros((uh.shape[0], _TBL_COLS - _NH_TOT - _NW_TOT), jnp.float32))
    out_ref[:] = jnp.concatenate(parts, axis=-1)


_FULL_COMPUTE = True

_LN2 = 0.6931471805599453
_SQRT2 = 1.4142135623730951


def _log_f32(y):
    bits = lax.bitcast_convert_type(y, jnp.int32)
    m = lax.bitcast_convert_type(
        jnp.bitwise_or(jnp.bitwise_and(bits, 0x007FFFFF), 0x3F800000), jnp.float32)
    e = jnp.right_shift(bits, 23) - 127
    big = m >= _SQRT2
    m = jnp.where(big, m * 0.5, m)
    ef = (e + jnp.where(big, 1, 0)).astype(jnp.float32)
    s = (m - 1.0) / (m + 1.0)
    s2 = s * s
    p = (1.0 / 9.0)
    p = p * s2 + (1.0 / 7.0)
    p = p * s2 + 0.2
    p = p * s2 + (1.0 / 3.0)
    p = p * s2 + 1.0
    return ef * _LN2 + 2.0 * s * p


def _splat_i32(v):
    return jnp.zeros((16,), jnp.int32) + v


def _level_multi(xs, cvecs, rows_v, delta_v, lvl):
    n = len(xs)
    nh = _NBINS[lvl]
    nb = nh - 1
    uh0 = _UH_OFF[lvl]
    w0 = _W_OFF[lvl]
    d0 = _UH_OFF[lvl]

    def e_at(cvec, k):
        cu = plsc.load_gather(rows_v, [cvec, _splat_i32(uh0 + k)])
        cd = plsc.load_gather(delta_v, [cvec, _splat_i32(d0 + k)])
        return jnp.exp(cu + cd)

    def w_at(cvec, j):
        return plsc.load_gather(rows_v, [cvec, _splat_i32(w0 + j)])

    zero = jnp.zeros((16,), jnp.float32)
    one = jnp.ones((16,), jnp.float32)

    def step(k, st, i, last):
        loc, area, cw, cloc, cle, cre, ccdf, ep = st
        e = e_at(cvecs[i], k)
        wj = w_at(cvecs[i], k - 1)
        t = (ep + e) * 0.5 * wj
        locn = loc + wj
        if last:
            sel = xs[i] >= loc
        else:
            sel = (xs[i] >= loc) & (xs[i] < locn)
        cw = jnp.where(sel, wj, cw)
        cloc = jnp.where(sel, loc, cloc)
        cle = jnp.where(sel, ep, cle)
        cre = jnp.where(sel, e, cre)
        ccdf = jnp.where(sel, area, ccdf)
        return (locn, area + t, cw, cloc, cle, cre, ccdf, e)

    def body(k, carry):
        out = []
        for i in range(n):
            out.extend(step(k, carry[8 * i:8 * i + 8], i, False))
        return tuple(out)

    carry = []
    for i in range(n):
        carry.extend((zero, zero, one, zero, one, one, zero, e_at(cvecs[i], 0)))
    carry = lax.fori_loop(1, nh - 1, body, tuple(carry), unroll=4)

    outs, lads = [], []
    for i in range(n):
        st = step(nh - 1, carry[8 * i:8 * i + 8], i, True)
        _, area, cw, cloc, cle, cre, ccdf, _ = st
        inv = 1.0 / area
        lh = cle * inv
        rh = cre * inv
        lcdf = ccdf * inv
        alpha = (xs[i] - cloc) / cw
        dh = rh - lh
        out = (0.5 * dh * cw) * alpha * alpha + (lh * cw) * alpha + lcdf
        outs.append(jnp.clip(out, 0.0, 1.0))
        lads.append(_log_f32(alpha * dh + lh))
    return outs, lads


def _sc_body(x_hbm, lgi_hbm, delta_hbm, table_hbm, genes_hbm,
             outx_hbm, outlad_hbm,
             genes_v, lgi_all, x_all, gidx_a, gidx_b,
             delta_a, delta_b, rows_a, rows_b, ox_all, ol_all, sem_a, sem_b):
    wid = lax.axis_index("s") * 2 + lax.axis_index("c")
    cuts_per = _N_CUTS // _NWORKERS
    ngroups = cuts_per // _B
    gidx_v = (gidx_a, gidx_b)
    delta_v = (delta_a, delta_b)
    rows_v = (rows_a, rows_b)
    sems = (sem_a, sem_b)
    tec_base = wid * cuts_per
    pltpu.sync_copy(genes_hbm, genes_v)
    pltpu.sync_copy(lgi_hbm.at[pl.ds(tec_base, cuts_per)], lgi_all)
    pltpu.sync_copy(x_hbm.at[pl.ds(tec_base, cuts_per)], x_all)
    lane = lax.iota(jnp.int32, 16)

    def prefetch(gi, b):
        base = tec_base + gi * _B
        for sub in range(_B // 16):
            g16 = lgi_all[pl.ds(gi * _B + sub * 16, 16)]
            gg = plsc.load_gather(genes_v, [jnp.right_shift(g16, 4),
                                            jnp.bitwise_and(g16, 15)])
            gidx_v[b][pl.ds(sub * 16, 16)] = plsc.bitcast(gg, jnp.int32)
        pltpu.async_copy(delta_hbm.at[pl.ds(base, _B)], delta_v[b], sems[b])
        pltpu.async_copy(table_hbm.at[gidx_v[b]], rows_v[b], sems[b])

    def wait_bufs(gi, b):
        base = tec_base + gi * _B
        pltpu.make_async_copy(delta_hbm.at[pl.ds(base, _B)], delta_v[b],
                              sems[b]).wait()
        pltpu.make_async_copy(table_hbm.at[gidx_v[b]], rows_v[b],
                              sems[b]).wait()

    prefetch(0, 0)

    def outer(gg, dummy):
        for b in range(2):
            gi = 2 * gg + b

            @pl.when(gi + 1 < ngroups)
            def _():
                prefetch(gi + 1, 1 - b)

            wait_bufs(gi, b)
            for pair in range(_B // 16):
                subs = (pair,)
                offs = [gi * _B + s * 16 for s in subs]
                cvecs = [lane + s * 16 for s in subs]
                xs = [x_all[pl.ds(off, 16)] for off in offs]
                lads = [jnp.zeros((16,), jnp.float32) for _ in subs]
                if _FULL_COMPUTE:
                    for lvl in range(3):
                        xs, ls = _level_multi(xs, cvecs, rows_v[b],
                                              delta_v[b], lvl)
                        lads = [a + c for a, c in zip(lads, ls)]
                else:
                    xs = [x + plsc.load_gather(rows_v[b], [cv, _splat_i32(0)])
                          for x, cv in zip(xs, cvecs)]
                    lads = [a + plsc.load_gather(delta_v[b], [cv, _splat_i32(0)])
                            for a, cv in zip(lads, cvecs)]
                for i, off in enumerate(offs):
                    ox_all[pl.ds(off, 16)] = xs[i]
                    ol_all[pl.ds(off, 16)] = lads[i]
        return dummy

    lax.fori_loop(0, ngroups // 2, outer, jnp.int32(0))
    pltpu.sync_copy(ox_all, outx_hbm.at[pl.ds(tec_base, cuts_per)])
    pltpu.sync_copy(ol_all, outlad_hbm.at[pl.ds(tec_base, cuts_per)])


@functools.lru_cache(maxsize=1)
def _get_sc_kernel():
    return functools.partial(
        pl.kernel,
        mesh=plsc.VectorSubcoreMesh(core_axis_name="c", subcore_axis_name="s"),
        compiler_params=pltpu.CompilerParams(use_tc_tiling_on_sc=False,
                                             needs_layout_passes=False),
        out_type=[jax.ShapeDtypeStruct((_N_CUTS,), jnp.float32),
                  jax.ShapeDtypeStruct((_N_CUTS,), jnp.float32)],
        scratch_types=[
            pltpu.VMEM((32, 16), jnp.float32),
            pltpu.VMEM((_N_CUTS // _NWORKERS,), jnp.int32),
            pltpu.VMEM((_N_CUTS // _NWORKERS,), jnp.float32),
            pltpu.VMEM((_B,), jnp.int32),
            pltpu.VMEM((_B,), jnp.int32),
            pltpu.VMEM((_B, _D_COLS), jnp.float32),
            pltpu.VMEM((_B, _D_COLS), jnp.float32),
            pltpu.VMEM((_B, _TBL_COLS), jnp.float32),
            pltpu.VMEM((_B, _TBL_COLS), jnp.float32),
            pltpu.VMEM((_N_CUTS // _NWORKERS,), jnp.float32),
            pltpu.VMEM((_N_CUTS // _NWORKERS,), jnp.float32),
            pltpu.SemaphoreType.DMA,
            pltpu.SemaphoreType.DMA,
        ],
    )(_sc_body)


def kernel(x, genes_oi, local_gene_ix, delta, unnormalized_heights, unnormalized_widths):
    uh = jnp.pad(unnormalized_heights, ((0, _NG_PAD - _N_GENES), (0, 0)))
    uw = jnp.pad(unnormalized_widths, ((0, _NG_PAD - _N_GENES), (0, _NH_TOT - _NW_TOT)))
    table = pl.pallas_call(
        _table_kernel,
        grid=(_NG_PAD // 512,),
        in_specs=[pl.BlockSpec((512, _NH_TOT), lambda i: (i, 0)),
                  pl.BlockSpec((512, _NH_TOT), lambda i: (i, 0))],
        out_specs=pl.BlockSpec((512, _TBL_COLS), lambda i: (i, 0)),
        out_shape=jax.ShapeDtypeStruct((_NG_PAD, _TBL_COLS), jnp.float32),
    )(uh, uw)
    genes_pad = lax.bitcast_convert_type(
        jnp.pad(genes_oi, (0, 512 - 500)).reshape(32, 16), jnp.float32)
    delta_p = jnp.pad(delta, ((0, 0), (0, _D_COLS - _NH_TOT)))
    outx, outlad = _get_sc_kernel()(x, local_gene_ix, delta_p, table, genes_pad)
    return outx, outlad
